# baseline (speedup 1.0000x reference)
import jax
import jax.numpy as jnp
from jax import lax
from jax.experimental import pallas as pl
from jax.experimental.pallas import tpu as pltpu
from jax.experimental.pallas import tpu_sc as plsc

B = 16384
NC = 2
NS = 16


def _body(x_hbm, tab_hbm, y_hbm, v, sem):
    wid = lax.axis_index("s") * NC + lax.axis_index("c")
    base = wid * 16
    pltpu.sync_copy(x_hbm.at[pl.ds(base, 16)], v)
    pltpu.sync_copy(v, y_hbm.at[pl.ds(base, 16)])


@jax.jit
def _probe(x, tab):
    mesh = plsc.VectorSubcoreMesh(
        core_axis_name="c", subcore_axis_name="s",
        num_cores=NC, num_subcores=NS)
    run = pl.kernel(
        _body,
        out_type=jax.ShapeDtypeStruct((B,), jnp.float32),
        mesh=mesh,
        scratch_types=[
            pltpu.VMEM((16,), jnp.float32),
            pltpu.SemaphoreType.DMA,
        ],
    )
    return run(x, tab)


def kernel(X, X_id, Z, P, alpha, beta):
    pair = jnp.concatenate(
        [alpha.astype(jnp.bfloat16), beta.astype(jnp.bfloat16)], axis=1)
    y = _probe(X[:, 0], pair)
    shp = X.shape
    return (y.reshape(shp), y.reshape(shp), y.reshape(shp))


# speedup vs baseline: 2.9382x; 2.9382x over previous
import jax
import jax.numpy as jnp
from jax import lax
from jax.experimental import pallas as pl
from jax.experimental.pallas import tpu as pltpu
from jax.experimental.pallas import tpu_sc as plsc

B = 16384
NC = 2
NS = 16
NW = NC * NS
CHUNK = B // NW
L = 16
GSLICE = 128
NG = CHUNK // GSLICE


_LN2 = 0.6931471805599453


def _decode_bf16(t):
    s = lax.shift_right_logical(t, jnp.int32(15))
    e = lax.shift_right_logical(t, jnp.int32(7)) & jnp.int32(0xFF)
    m = t & jnp.int32(0x7F)
    mf = m.astype(jnp.float32)
    mant = jnp.where(e == 0, mf * (1.0 / 64.0), mf * (1.0 / 128.0) + 1.0)
    val = mant * jnp.exp((e - 127).astype(jnp.float32) * _LN2)
    return jnp.where(s == 1, -val, val)


def _tsbrnn_body(x_hbm, xid_hbm, z_hbm, p_hbm, pair_hbm,
                 y_hbm, zn_hbm, pn_hbm,
                 idx_v, pr_v, x_v, z_v, p_v,
                 y_v, zn_v, pn_v, sem_g, sem_s, sem_o):
    wid = lax.axis_index("s") * NC + lax.axis_index("c")
    base = wid * CHUNK
    blk = pl.ds(base, CHUNK)

    pltpu.sync_copy(xid_hbm.at[blk], idx_v)
    gathers = []
    for g in range(NG):
        sl = pl.ds(g * GSLICE, GSLICE)
        gathers.append(pltpu.async_copy(pair_hbm.at[idx_v.at[sl]], pr_v.at[sl], sem_g))
    stages = [pltpu.async_copy(x_hbm.at[blk], x_v, sem_s),
              pltpu.async_copy(z_hbm.at[blk], z_v, sem_s),
              pltpu.async_copy(p_hbm.at[blk], p_v, sem_s)]
    for cp in stages:
        cp.wait()
    for cp in gathers:
        cp.wait()

    for i in range(CHUNK // L):
        sl = pl.ds(i * L, L)
        t = pr_v[sl]
        a = _decode_bf16(lax.shift_right_logical(t, jnp.int32(16)))
        b = _decode_bf16(t & jnp.int32(0xFFFF))
        x = x_v[sl]
        z = z_v[sl]
        p = p_v[sl]
        nz = x != 0.0
        zn = jnp.where(nz, a * x + (1.0 - a) * z, z)
        pn = jnp.where(nz, b, 0.0) + (1.0 - b) * p
        y_v[sl] = zn * pn
        zn_v[sl] = zn
        pn_v[sl] = pn

    outs = [pltpu.async_copy(y_v, y_hbm.at[blk], sem_o),
            pltpu.async_copy(zn_v, zn_hbm.at[blk], sem_o),
            pltpu.async_copy(pn_v, pn_hbm.at[blk], sem_o)]
    for cp in outs:
        cp.wait()


@jax.jit
def _tsbrnn(x, xid, z, p, pair):
    mesh = plsc.VectorSubcoreMesh(
        core_axis_name="c", subcore_axis_name="s",
        num_cores=NC, num_subcores=NS)
    vec = jax.ShapeDtypeStruct((B,), jnp.float32)
    run = pl.kernel(
        _tsbrnn_body,
        out_type=(vec, vec, vec),
        mesh=mesh,
        scratch_types=[
            pltpu.VMEM((CHUNK,), jnp.int32),
            pltpu.VMEM((CHUNK,), jnp.int32),
            pltpu.VMEM((CHUNK,), jnp.float32),
            pltpu.VMEM((CHUNK,), jnp.float32),
            pltpu.VMEM((CHUNK,), jnp.float32),
            pltpu.VMEM((CHUNK,), jnp.float32),
            pltpu.VMEM((CHUNK,), jnp.float32),
            pltpu.VMEM((CHUNK,), jnp.float32),
            pltpu.SemaphoreType.DMA,
            pltpu.SemaphoreType.DMA,
            pltpu.SemaphoreType.DMA,
        ],
    )
    return run(x, xid, z, p, pair)


def kernel(X, X_id, Z, P, alpha, beta):
    a32 = lax.bitcast_convert_type(alpha[:, 0], jnp.uint32)
    b32 = lax.bitcast_convert_type(beta[:, 0], jnp.uint32)
    sh = jnp.uint32(16)
    a16 = lax.shift_right_logical(a32 + jnp.uint32(0x8000), sh)
    b16 = lax.shift_right_logical(b32 + jnp.uint32(0x8000), sh)
    pair = lax.bitcast_convert_type(
        lax.shift_left(a16, sh) | b16, jnp.int32)
    y, zn, pn = _tsbrnn(X[:, 0], X_id[:, 0], Z[:, 0], P[:, 0], pair)
    shp = X.shape
    return (y.reshape(shp), zn.reshape(shp), pn.reshape(shp))
